# baseline (speedup 1.0000x reference)
import functools
import math

import jax
import jax.numpy as jnp
from jax import lax
from jax.experimental import pallas as pl
from jax.experimental.pallas import tpu as pltpu
from jax.experimental.pallas import tpu_sc as plsc

D_MODEL_K = 128
VOCAB_K = 100000
SCALE = math.sqrt(D_MODEL_K)

_info = plsc.get_sparse_core_info()
_NC, _NS, _L = _info.num_cores, _info.num_subcores, _info.num_lanes
_NW = _NC * _NS

_GROUP = 128
_NBUF = 5
_LA = 2


def _make_sc_gather(n_idx: int):
    assert n_idx % (_NW * _GROUP * _NBUF) == 0
    per_w = n_idx // _NW
    n_groups = per_w // _GROUP
    n_steps = n_groups // _NBUF

    mesh = plsc.VectorSubcoreMesh(core_axis_name="c", subcore_axis_name="s")

    @functools.partial(
        pl.kernel,
        mesh=mesh,
        out_type=jax.ShapeDtypeStruct((n_idx, D_MODEL_K), jnp.float32),
        scratch_types=[
            pltpu.VMEM((n_groups, _GROUP), jnp.int32),
            pltpu.VMEM((_NBUF, _GROUP, D_MODEL_K), jnp.float32),
        ] + [pltpu.SemaphoreType.DMA] * (2 * _NBUF),
    )
    def sc_gather(idx_hbm, table_hbm, out_hbm, idx_v, bufs, *sems):
        sin = sems[:_NBUF]
        sout = sems[_NBUF:]
        wid = lax.axis_index("s") * _NC + lax.axis_index("c")
        base = wid * per_w
        pltpu.sync_copy(idx_hbm.at[wid], idx_v)

        def in_start(g, b):
            pltpu.async_copy(
                table_hbm.at[pl.ds(wid * 1024 + (g % 8) * _GROUP, _GROUP)],
                bufs.at[b], sin[b])

        def in_wait(g, b):
            pltpu.make_async_copy(
                table_hbm.at[pl.ds(wid * 1024 + (g % 8) * _GROUP, _GROUP)],
                bufs.at[b], sin[b]).wait()

        def out_start(g, b):
            pltpu.async_copy(bufs.at[b], out_hbm.at[idx_v.at[g]], sout[b])

        def out_wait(g, b):
            pltpu.make_async_copy(bufs.at[b], out_hbm.at[idx_v.at[g]],
                                  sout[b]).wait()

        for j in range(_LA):
            in_start(j, j)

        def step_body(s, carry):
            for b in range(_NBUF):
                g = s * _NBUF + b
                nb = (b + _LA) % _NBUF

                @pl.when(g + _LA - _NBUF >= 0)
                def _():
                    out_wait(g + _LA - _NBUF, nb)

                @pl.when(g + _LA < n_groups)
                def _():
                    in_start(g + _LA, nb)

                in_wait(g, b)
                out_start(g, b)
            return carry

        lax.fori_loop(0, n_steps, step_body, 0, unroll=False)

        for j in range(_NBUF - _LA):
            g = n_groups - (_NBUF - _LA) + j
            out_wait(g, g % _NBUF)

    return sc_gather


def kernel(x, lut):
    b, s = x.shape
    n = b * s
    idx = x.reshape(_NW, n // (_NW * _GROUP), _GROUP).astype(jnp.int32)
    out = _make_sc_gather(n)(idx, lut)
    return out.reshape(b, s, D_MODEL_K)


# speedup vs baseline: 1.0043x; 1.0043x over previous
import functools
import math

import jax
import jax.numpy as jnp
from jax import lax
from jax.experimental import pallas as pl
from jax.experimental.pallas import tpu as pltpu
from jax.experimental.pallas import tpu_sc as plsc

D_MODEL_K = 128
VOCAB_K = 100000
SCALE = math.sqrt(D_MODEL_K)

_info = plsc.get_sparse_core_info()
_NC, _NS, _L = _info.num_cores, _info.num_subcores, _info.num_lanes
_NW = _NC * _NS

_GROUP = 128
_NBUF = 5
_LA = 2


def _make_sc_gather(n_idx: int):
    assert n_idx % (_NW * _GROUP * _NBUF) == 0
    per_w = n_idx // _NW
    n_groups = per_w // _GROUP
    n_steps = n_groups // _NBUF

    mesh = plsc.VectorSubcoreMesh(core_axis_name="c", subcore_axis_name="s")

    @functools.partial(
        pl.kernel,
        mesh=mesh,
        out_type=jax.ShapeDtypeStruct((n_idx, D_MODEL_K), jnp.float32),
        scratch_types=[
            pltpu.VMEM((n_groups, _GROUP), jnp.int32),
            pltpu.VMEM((_NBUF, _GROUP, D_MODEL_K), jnp.float32),
        ] + [pltpu.SemaphoreType.DMA] * (2 * _NBUF),
    )
    def sc_gather(idx_hbm, table_hbm, out_hbm, idx_v, bufs, *sems):
        sin = sems[:_NBUF]
        sout = sems[_NBUF:]
        wid = lax.axis_index("s") * _NC + lax.axis_index("c")
        base = wid * per_w
        pltpu.sync_copy(idx_hbm.at[wid], idx_v)

        def in_start(g, b):
            pltpu.async_copy(table_hbm.at[idx_v.at[g]], bufs.at[b], sin[b])

        def in_wait(g, b):
            pltpu.make_async_copy(table_hbm.at[idx_v.at[g]], bufs.at[b],
                                  sin[b]).wait()

        def out_start(g, b):
            pltpu.async_copy(bufs.at[b],
                             out_hbm.at[pl.ds(base + g * _GROUP, _GROUP)],
                             sout[b])

        def out_wait(b):
            pltpu.make_async_copy(bufs.at[b],
                                  out_hbm.at[pl.ds(base, _GROUP)],
                                  sout[b]).wait()

        for j in range(_LA):
            in_start(j, j)

        def step_body(s, carry):
            for b in range(_NBUF):
                g = s * _NBUF + b
                nb = (b + _LA) % _NBUF

                @pl.when(g + _LA - _NBUF >= 0)
                def _():
                    out_wait(nb)

                @pl.when(g + _LA < n_groups)
                def _():
                    in_start(g + _LA, nb)

                in_wait(g, b)

                def row_body(r, c2):
                    for c in range(D_MODEL_K // _L):
                        sl = (b, r, pl.ds(c * _L, _L))
                        bufs[sl] = bufs[sl] * SCALE
                    return c2

                lax.fori_loop(0, _GROUP, row_body, 0, unroll=2)

                out_start(g, b)
            return carry

        lax.fori_loop(0, n_steps, step_body, 0, unroll=False)

        for j in range(_NBUF - _LA):
            out_wait((n_groups - (_NBUF - _LA) + j) % _NBUF)

    return sc_gather


def kernel(x, lut):
    b, s = x.shape
    n = b * s
    idx = x.reshape(_NW, n // (_NW * _GROUP), _GROUP).astype(jnp.int32)
    out = _make_sc_gather(n)(idx, lut)
    return out.reshape(b, s, D_MODEL_K)
